# speedup vs baseline: 1.4704x; 1.4704x over previous
import functools

import jax
import jax.numpy as jnp
from jax import lax
from jax.experimental import pallas as pl
from jax.experimental.pallas import tpu as pltpu
from jax.experimental.pallas import tpu_sc as plsc

_GROUPS = ((1, 2), (3, 4), (5, 6), (7, 8), (0, 9),
           (10, 11, 12), (13, 14), (15, 16), (17, 18), (19, 20))
_J = 21
_G = 10
_C = 128
_LANES = 16
_NW = 32
_R = 8


def _make_kernel(N):
    rows_per_w = N // _NW
    steps = rows_per_w // _R
    mesh = plsc.VectorSubcoreMesh(core_axis_name="c", subcore_axis_name="s")

    @functools.partial(
        pl.kernel,
        out_type=jax.ShapeDtypeStruct((N, _G * _C), jnp.float32),
        mesh=mesh,
        scratch_types=[
            pltpu.VMEM((_R, _J * _C), jnp.float32),
            pltpu.VMEM((_R, _G * _C), jnp.float32),
        ],
    )
    def k(x_hbm, out_hbm, in_v, out_v):
        wid = lax.axis_index("s") * 2 + lax.axis_index("c")
        base = wid * rows_per_w

        @pl.loop(0, steps)
        def _step(step):
            row0 = base + step * _R
            pltpu.sync_copy(x_hbm.at[pl.ds(row0, _R)], in_v)
            for r in range(_R):
                for s in range(_C // _LANES):
                    off = s * _LANES
                    vals = [in_v[r, pl.ds(j * _C + off, _LANES)]
                            for j in range(_J)]
                    for g, grp in enumerate(_GROUPS):
                        acc = vals[grp[0]]
                        for j in grp[1:]:
                            acc = acc + vals[j]
                        out_v[r, pl.ds(g * _C + off, _LANES)] = (
                            acc * (1.0 / len(grp)))
            pltpu.sync_copy(out_v, out_hbm.at[pl.ds(row0, _R)])

    return k


def kernel(data2):
    B, T, J, C = data2.shape
    N = B * T
    x = data2.reshape(N, J * C)
    out = _make_kernel(N)(x)
    return out.reshape(B, T, _G, C)


# speedup vs baseline: 1.7339x; 1.1793x over previous
import functools

import jax
import jax.numpy as jnp
from jax import lax
from jax.experimental import pallas as pl
from jax.experimental.pallas import tpu as pltpu
from jax.experimental.pallas import tpu_sc as plsc

_GROUPS = ((1, 2), (3, 4), (5, 6), (7, 8), (0, 9),
           (10, 11, 12), (13, 14), (15, 16), (17, 18), (19, 20))
_J = 21
_G = 10
_C = 128
_LANES = 16
_NW = 32
_R = 8


_NBUF = 2


def _make_kernel(N):
    rows_per_w = N // _NW
    steps = rows_per_w // _R
    assert steps % _NBUF == 0
    mesh = plsc.VectorSubcoreMesh(core_axis_name="c", subcore_axis_name="s")

    @functools.partial(
        pl.kernel,
        out_type=jax.ShapeDtypeStruct((N, _G * _C), jnp.float32),
        mesh=mesh,
        scratch_types=[
            pltpu.VMEM((_NBUF, _R, _J * _C), jnp.float32),
            pltpu.VMEM((_NBUF, _R, _G * _C), jnp.float32),
            pltpu.SemaphoreType.DMA,
            pltpu.SemaphoreType.DMA,
            pltpu.SemaphoreType.DMA,
            pltpu.SemaphoreType.DMA,
        ],
    )
    def k(x_hbm, out_hbm, in_v, out_v, si0, si1, so0, so1):
        sin = (si0, si1)
        sout = (so0, so1)
        wid = lax.axis_index("s") * 2 + lax.axis_index("c")
        base = wid * rows_per_w

        def in_copy(step, b):
            return pltpu.make_async_copy(
                x_hbm.at[pl.ds(base + step * _R, _R)], in_v.at[b], sin[b])

        def out_copy(step, b):
            return pltpu.make_async_copy(
                out_v.at[b], out_hbm.at[pl.ds(base + step * _R, _R)], sout[b])

        for b in range(_NBUF):
            in_copy(b, b).start()

        @pl.loop(0, steps, step=_NBUF)
        def _block(g):
            for b in range(_NBUF):
                step = g + b
                in_copy(step, b).wait()

                @pl.when(step >= _NBUF)
                def _():
                    out_copy(step - _NBUF, b).wait()

                for r in range(_R):
                    for s in range(_C // _LANES):
                        off = s * _LANES
                        vals = [in_v[b, r, pl.ds(j * _C + off, _LANES)]
                                for j in range(_J)]
                        for gi, grp in enumerate(_GROUPS):
                            acc = vals[grp[0]]
                            for j in grp[1:]:
                                acc = acc + vals[j]
                            out_v[b, r, pl.ds(gi * _C + off, _LANES)] = (
                                acc * (1.0 / len(grp)))

                out_copy(step, b).start()

                @pl.when(step + _NBUF < steps)
                def _():
                    in_copy(step + _NBUF, b).start()

        for b in range(_NBUF):
            out_copy(steps - _NBUF + b, b).wait()

    return k


def kernel(data2):
    B, T, J, C = data2.shape
    N = B * T
    x = data2.reshape(N, J * C)
    out = _make_kernel(N)(x)
    return out.reshape(B, T, _G, C)


# speedup vs baseline: 2.7273x; 1.5729x over previous
import functools

import jax
import jax.numpy as jnp
from jax import lax
from jax.experimental import pallas as pl
from jax.experimental.pallas import tpu as pltpu
from jax.experimental.pallas import tpu_sc as plsc

_GROUPS = ((1, 2), (3, 4), (5, 6), (7, 8), (0, 9),
           (10, 11, 12), (13, 14), (15, 16), (17, 18), (19, 20))
_J = 21
_G = 10
_C = 128
_LANES = 16
_NW = 32
_R = 8
_NBUF = 2


def _make_kernel(N):
    rows_per_w = N // _NW
    steps = rows_per_w // _R
    assert steps % _NBUF == 0
    mesh = plsc.VectorSubcoreMesh(core_axis_name="c", subcore_axis_name="s")

    @functools.partial(
        pl.kernel,
        out_type=jax.ShapeDtypeStruct((N, _G, _C), jnp.float32),
        mesh=mesh,
        scratch_types=[
            pltpu.VMEM((_NBUF, _R, _J, _C), jnp.float32),
            pltpu.VMEM((_NBUF, _R, _G, _C), jnp.float32),
            pltpu.SemaphoreType.DMA,
            pltpu.SemaphoreType.DMA,
            pltpu.SemaphoreType.DMA,
            pltpu.SemaphoreType.DMA,
        ],
        compiler_params=pltpu.CompilerParams(use_tc_tiling_on_sc=True),
    )
    def k(x_hbm, out_hbm, in_v, out_v, si0, si1, so0, so1):
        sin = (si0, si1)
        sout = (so0, so1)
        wid = lax.axis_index("s") * 2 + lax.axis_index("c")
        base = wid * rows_per_w

        def in_copy(step, b):
            return pltpu.make_async_copy(
                x_hbm.at[pl.ds(base + step * _R, _R)], in_v.at[b], sin[b])

        def out_copy(step, b):
            return pltpu.make_async_copy(
                out_v.at[b], out_hbm.at[pl.ds(base + step * _R, _R)], sout[b])

        for b in range(_NBUF):
            in_copy(b, b).start()

        @pl.loop(0, steps, step=_NBUF)
        def _block(g):
            for b in range(_NBUF):
                step = g + b
                in_copy(step, b).wait()

                @pl.when(step >= _NBUF)
                def _():
                    out_copy(step - _NBUF, b).wait()

                for r in range(_R):
                    for s in range(_C // _LANES):
                        off = s * _LANES
                        vals = [in_v[b, r, j, pl.ds(off, _LANES)]
                                for j in range(_J)]
                        for gi, grp in enumerate(_GROUPS):
                            acc = vals[grp[0]]
                            for j in grp[1:]:
                                acc = acc + vals[j]
                            out_v[b, r, gi, pl.ds(off, _LANES)] = (
                                acc * (1.0 / len(grp)))

                out_copy(step, b).start()

                @pl.when(step + _NBUF < steps)
                def _():
                    in_copy(step + _NBUF, b).start()

        for b in range(_NBUF):
            out_copy(steps - _NBUF + b, b).wait()

    return k


def kernel(data2):
    B, T, J, C = data2.shape
    N = B * T
    x = data2.reshape(N, J, C)
    out = _make_kernel(N)(x)
    return out.reshape(B, T, _G, C)


# speedup vs baseline: 2.8072x; 1.0293x over previous
import functools

import jax
import jax.numpy as jnp
from jax import lax
from jax.experimental import pallas as pl
from jax.experimental.pallas import tpu as pltpu
from jax.experimental.pallas import tpu_sc as plsc

_GROUPS = ((1, 2), (3, 4), (5, 6), (7, 8), (0, 9),
           (10, 11, 12), (13, 14), (15, 16), (17, 18), (19, 20))
_J = 21
_G = 10
_C = 128
_LANES = 16
_NW = 32
_R = 8
_NBUF = 2


def _make_kernel(B, T):
    b_per_w = B // _NW
    t_steps = T // _R
    steps = b_per_w * t_steps
    assert steps % _NBUF == 0
    mesh = plsc.VectorSubcoreMesh(core_axis_name="c", subcore_axis_name="s")

    @functools.partial(
        pl.kernel,
        out_type=jax.ShapeDtypeStruct((B, T, _G, _C), jnp.float32),
        mesh=mesh,
        scratch_types=[
            pltpu.VMEM((_NBUF, _R, _J, _C), jnp.float32),
            pltpu.VMEM((_NBUF, _R, _G, _C), jnp.float32),
            pltpu.SemaphoreType.DMA,
            pltpu.SemaphoreType.DMA,
            pltpu.SemaphoreType.DMA,
            pltpu.SemaphoreType.DMA,
        ],
        compiler_params=pltpu.CompilerParams(use_tc_tiling_on_sc=True),
    )
    def k(x_hbm, out_hbm, in_v, out_v, si0, si1, so0, so1):
        sin = (si0, si1)
        sout = (so0, so1)
        wid = lax.axis_index("s") * 2 + lax.axis_index("c")
        base_b = wid * b_per_w

        def in_copy(step, b):
            bb = base_b + step // t_steps
            t0 = (step % t_steps) * _R
            return pltpu.make_async_copy(
                x_hbm.at[bb, pl.ds(t0, _R)], in_v.at[b], sin[b])

        def out_copy(step, b):
            bb = base_b + step // t_steps
            t0 = (step % t_steps) * _R
            return pltpu.make_async_copy(
                out_v.at[b], out_hbm.at[bb, pl.ds(t0, _R)], sout[b])

        for b in range(_NBUF):
            in_copy(b, b).start()

        @pl.loop(0, steps, step=_NBUF)
        def _block(g):
            for b in range(_NBUF):
                step = g + b
                in_copy(step, b).wait()

                @pl.when(step >= _NBUF)
                def _():
                    out_copy(step - _NBUF, b).wait()

                for r in range(_R):
                    for s in range(_C // _LANES):
                        off = s * _LANES
                        vals = [in_v[b, r, j, pl.ds(off, _LANES)]
                                for j in range(_J)]
                        for gi, grp in enumerate(_GROUPS):
                            acc = vals[grp[0]]
                            for j in grp[1:]:
                                acc = acc + vals[j]
                            out_v[b, r, gi, pl.ds(off, _LANES)] = (
                                acc * (1.0 / len(grp)))

                out_copy(step, b).start()

                @pl.when(step + _NBUF < steps)
                def _():
                    in_copy(step + _NBUF, b).start()

        for b in range(_NBUF):
            out_copy(steps - _NBUF + b, b).wait()

    return k


def kernel(data2):
    B, T, J, C = data2.shape
    return _make_kernel(B, T)(data2)


# speedup vs baseline: 5.1180x; 1.8231x over previous
import functools

import jax
import jax.numpy as jnp
from jax import lax
from jax.experimental import pallas as pl
from jax.experimental.pallas import tpu as pltpu
from jax.experimental.pallas import tpu_sc as plsc

_GROUPS = ((1, 2), (3, 4), (5, 6), (7, 8), (0, 9),
           (10, 11, 12), (13, 14), (15, 16), (17, 18), (19, 20))
_J = 21
_G = 10
_C = 128
_LANES = 16
_NW = 32
_R = 8
_NBUF = 2


def _make_kernel(B, T):
    b_per_w = B // _NW
    t_steps = T // _R
    steps = b_per_w * t_steps
    assert steps % _NBUF == 0
    mesh = plsc.VectorSubcoreMesh(core_axis_name="c", subcore_axis_name="s")

    @functools.partial(
        pl.kernel,
        out_type=jax.ShapeDtypeStruct((B, _G, T, _C), jnp.float32),
        mesh=mesh,
        scratch_types=[
            pltpu.VMEM((_NBUF, _J, _R, _C), jnp.float32),
            pltpu.VMEM((_NBUF, _G, _R, _C), jnp.float32),
            pltpu.SemaphoreType.DMA,
            pltpu.SemaphoreType.DMA,
            pltpu.SemaphoreType.DMA,
            pltpu.SemaphoreType.DMA,
        ],
    )
    def k(x_hbm, out_hbm, in_v, out_v, si0, si1, so0, so1):
        sin = (si0, si1)
        sout = (so0, so1)
        wid = lax.axis_index("s") * 2 + lax.axis_index("c")
        base_b = wid * b_per_w

        def in_copy(step, b):
            bb = base_b + step // t_steps
            t0 = (step % t_steps) * _R
            return pltpu.make_async_copy(
                x_hbm.at[bb, :, pl.ds(t0, _R)], in_v.at[b], sin[b])

        def out_copy(step, b):
            bb = base_b + step // t_steps
            t0 = (step % t_steps) * _R
            return pltpu.make_async_copy(
                out_v.at[b], out_hbm.at[bb, :, pl.ds(t0, _R)], sout[b])

        for b in range(_NBUF):
            in_copy(b, b).start()

        @pl.loop(0, steps, step=_NBUF)
        def _block(g):
            for b in range(_NBUF):
                step = g + b
                in_copy(step, b).wait()

                @pl.when(step >= _NBUF)
                def _():
                    out_copy(step - _NBUF, b).wait()

                for r in range(_R):
                    for s in range(_C // _LANES):
                        off = s * _LANES
                        vals = [in_v[b, j, r, pl.ds(off, _LANES)]
                                for j in range(_J)]
                        for gi, grp in enumerate(_GROUPS):
                            acc = vals[grp[0]]
                            for j in grp[1:]:
                                acc = acc + vals[j]
                            out_v[b, gi, r, pl.ds(off, _LANES)] = (
                                acc * (1.0 / len(grp)))

                out_copy(step, b).start()

                @pl.when(step + _NBUF < steps)
                def _():
                    in_copy(step + _NBUF, b).start()

        for b in range(_NBUF):
            out_copy(steps - _NBUF + b, b).wait()

    return k


def kernel(data2):
    B, T, J, C = data2.shape
    x = jnp.transpose(data2, (0, 2, 1, 3))
    out = _make_kernel(B, T)(x)
    return jnp.transpose(out, (0, 2, 1, 3))


# speedup vs baseline: 8.3967x; 1.6406x over previous
import functools

import jax
import jax.numpy as jnp
from jax import lax
from jax.experimental import pallas as pl
from jax.experimental.pallas import tpu as pltpu
from jax.experimental.pallas import tpu_sc as plsc

_GROUPS = ((1, 2), (3, 4), (5, 6), (7, 8), (0, 9),
           (10, 11, 12), (13, 14), (15, 16), (17, 18), (19, 20))
_J = 21
_G = 10
_C = 128
_LANES = 16
_NW = 32
_R = 8
_NBUF = 2


def _make_kernel(B, T):
    b_per_w = B // _NW
    t_steps = T // _R
    steps = b_per_w * t_steps
    assert steps % _NBUF == 0
    mesh = plsc.VectorSubcoreMesh(core_axis_name="c", subcore_axis_name="s")

    @functools.partial(
        pl.kernel,
        out_type=jax.ShapeDtypeStruct((B, _G, T, _C), jnp.float32),
        mesh=mesh,
        scratch_types=[
            pltpu.VMEM((_NBUF, _J, _R, _C), jnp.float32),
            pltpu.VMEM((_NBUF, _G, _R, _C), jnp.float32),
            pltpu.SemaphoreType.DMA,
            pltpu.SemaphoreType.DMA,
            pltpu.SemaphoreType.DMA,
            pltpu.SemaphoreType.DMA,
        ],
    )
    def k(x_hbm, out_hbm, in_v, out_v, si0, si1, so0, so1):
        sin = (si0, si1)
        sout = (so0, so1)
        wid = lax.axis_index("s") * 2 + lax.axis_index("c")
        base_b = wid * b_per_w

        def in_copy(step, b):
            bb = base_b + step // t_steps
            t0 = (step % t_steps) * _R
            return pltpu.make_async_copy(
                x_hbm.at[bb, :, pl.ds(t0, _R)], in_v.at[b], sin[b])

        def out_copy(step, b):
            bb = base_b + step // t_steps
            t0 = (step % t_steps) * _R
            return pltpu.make_async_copy(
                out_v.at[b], out_hbm.at[bb, :, pl.ds(t0, _R)], sout[b])

        for b in range(_NBUF):
            in_copy(b, b).start()

        @pl.loop(0, steps, step=_NBUF)
        def _block(g):
            for b in range(_NBUF):
                step = g + b
                in_copy(step, b).wait()

                @pl.when(step >= _NBUF)
                def _():
                    out_copy(step - _NBUF, b).wait()

                @pl.loop(0, _R)
                def _row(r):
                    for s in range(_C // _LANES):
                        off = s * _LANES
                        for gi, grp in enumerate(_GROUPS):
                            acc = in_v[b, grp[0], r, pl.ds(off, _LANES)]
                            for j in grp[1:]:
                                acc = acc + in_v[b, j, r, pl.ds(off, _LANES)]
                            out_v[b, gi, r, pl.ds(off, _LANES)] = (
                                acc * (1.0 / len(grp)))

                out_copy(step, b).start()

                @pl.when(step + _NBUF < steps)
                def _():
                    in_copy(step + _NBUF, b).start()

        for b in range(_NBUF):
            out_copy(steps - _NBUF + b, b).wait()

    return k


def kernel(data2):
    B, T, J, C = data2.shape
    x = jnp.transpose(data2, (0, 2, 1, 3))
    out = _make_kernel(B, T)(x)
    return jnp.transpose(out, (0, 2, 1, 3))


# speedup vs baseline: 9.1237x; 1.0866x over previous
import functools

import jax
import jax.numpy as jnp
from jax import lax
from jax.experimental import pallas as pl
from jax.experimental.pallas import tpu as pltpu
from jax.experimental.pallas import tpu_sc as plsc

_GROUPS = ((1, 2), (3, 4), (5, 6), (7, 8), (0, 9),
           (10, 11, 12), (13, 14), (15, 16), (17, 18), (19, 20))
_J = 21
_G = 10
_C = 128
_LANES = 16
_NW = 32
_R = 16
_NBUF = 2


def _make_kernel(B, T):
    b_per_w = B // _NW
    t_steps = T // _R
    steps = b_per_w * t_steps
    assert steps % _NBUF == 0
    mesh = plsc.VectorSubcoreMesh(core_axis_name="c", subcore_axis_name="s")

    @functools.partial(
        pl.kernel,
        out_type=jax.ShapeDtypeStruct((B, _G, T, _C), jnp.float32),
        mesh=mesh,
        scratch_types=[
            pltpu.VMEM((_NBUF, _J, _R, _C), jnp.float32),
            pltpu.VMEM((_NBUF, _G, _R, _C), jnp.float32),
            pltpu.SemaphoreType.DMA,
            pltpu.SemaphoreType.DMA,
            pltpu.SemaphoreType.DMA,
            pltpu.SemaphoreType.DMA,
        ],
    )
    def k(x_hbm, out_hbm, in_v, out_v, si0, si1, so0, so1):
        sin = (si0, si1)
        sout = (so0, so1)
        wid = lax.axis_index("s") * 2 + lax.axis_index("c")
        base_b = wid * b_per_w

        def in_copy(step, b):
            bb = base_b + step // t_steps
            t0 = (step % t_steps) * _R
            return pltpu.make_async_copy(
                x_hbm.at[bb, :, pl.ds(t0, _R)], in_v.at[b], sin[b])

        def out_copy(step, b):
            bb = base_b + step // t_steps
            t0 = (step % t_steps) * _R
            return pltpu.make_async_copy(
                out_v.at[b], out_hbm.at[bb, :, pl.ds(t0, _R)], sout[b])

        for b in range(_NBUF):
            in_copy(b, b).start()

        @pl.loop(0, steps, step=_NBUF)
        def _block(g):
            for b in range(_NBUF):
                step = g + b
                in_copy(step, b).wait()

                @pl.when(step >= _NBUF)
                def _():
                    out_copy(step - _NBUF, b).wait()

                @pl.loop(0, _R)
                def _row(r):
                    for s in range(_C // _LANES):
                        off = s * _LANES
                        for gi, grp in enumerate(_GROUPS):
                            acc = in_v[b, grp[0], r, pl.ds(off, _LANES)]
                            for j in grp[1:]:
                                acc = acc + in_v[b, j, r, pl.ds(off, _LANES)]
                            out_v[b, gi, r, pl.ds(off, _LANES)] = (
                                acc * (1.0 / len(grp)))

                out_copy(step, b).start()

                @pl.when(step + _NBUF < steps)
                def _():
                    in_copy(step + _NBUF, b).start()

        for b in range(_NBUF):
            out_copy(steps - _NBUF + b, b).wait()

    return k


def kernel(data2):
    B, T, J, C = data2.shape
    x = jnp.transpose(data2, (0, 2, 1, 3))
    out = _make_kernel(B, T)(x)
    return jnp.transpose(out, (0, 2, 1, 3))


# speedup vs baseline: 11.0695x; 1.2133x over previous
import functools

import jax
import jax.numpy as jnp
from jax import lax
from jax.experimental import pallas as pl
from jax.experimental.pallas import tpu as pltpu
from jax.experimental.pallas import tpu_sc as plsc

_GROUPS = ((1, 2), (3, 4), (5, 6), (7, 8), (0, 9),
           (10, 11, 12), (13, 14), (15, 16), (17, 18), (19, 20))
_J = 21
_G = 10
_C = 128
_LANES = 16
_NW = 32
_R = 8
_NBUF = 4


def _make_kernel(B, T):
    b_per_w = B // _NW
    t_steps = T // _R
    steps = b_per_w * t_steps
    assert steps % _NBUF == 0
    mesh = plsc.VectorSubcoreMesh(core_axis_name="c", subcore_axis_name="s")

    @functools.partial(
        pl.kernel,
        out_type=jax.ShapeDtypeStruct((B, _G, T, _C), jnp.float32),
        mesh=mesh,
        scratch_types=[
            pltpu.VMEM((_NBUF, _J, _R, _C), jnp.float32),
            pltpu.VMEM((_NBUF, _G, _R, _C), jnp.float32),
        ] + [pltpu.SemaphoreType.DMA] * (2 * _NBUF),
    )
    def k(x_hbm, out_hbm, in_v, out_v, *sems):
        sin = sems[:_NBUF]
        sout = sems[_NBUF:]
        wid = lax.axis_index("s") * 2 + lax.axis_index("c")
        base_b = wid * b_per_w

        def in_copy(step, b):
            bb = base_b + step // t_steps
            t0 = (step % t_steps) * _R
            return pltpu.make_async_copy(
                x_hbm.at[bb, :, pl.ds(t0, _R)], in_v.at[b], sin[b])

        def out_copy(step, b):
            bb = base_b + step // t_steps
            t0 = (step % t_steps) * _R
            return pltpu.make_async_copy(
                out_v.at[b], out_hbm.at[bb, :, pl.ds(t0, _R)], sout[b])

        for b in range(_NBUF):
            in_copy(b, b).start()

        @pl.loop(0, steps, step=_NBUF)
        def _block(g):
            for b in range(_NBUF):
                step = g + b
                in_copy(step, b).wait()

                @pl.when(step >= _NBUF)
                def _():
                    out_copy(step - _NBUF, b).wait()

                @pl.loop(0, _R)
                def _row(r):
                    for s in range(_C // _LANES):
                        off = s * _LANES
                        for gi, grp in enumerate(_GROUPS):
                            acc = in_v[b, grp[0], r, pl.ds(off, _LANES)]
                            for j in grp[1:]:
                                acc = acc + in_v[b, j, r, pl.ds(off, _LANES)]
                            out_v[b, gi, r, pl.ds(off, _LANES)] = (
                                acc * (1.0 / len(grp)))

                out_copy(step, b).start()

                @pl.when(step + _NBUF < steps)
                def _():
                    in_copy(step + _NBUF, b).start()

        for b in range(_NBUF):
            out_copy(steps - _NBUF + b, b).wait()

    return k


def kernel(data2):
    B, T, J, C = data2.shape
    x = jnp.transpose(data2, (0, 2, 1, 3))
    out = _make_kernel(B, T)(x)
    return jnp.transpose(out, (0, 2, 1, 3))
